# baseline (device time: 180930 ns/iter reference)
import os

import jax
import jax.numpy as jnp
from jax import lax
from jax.experimental import pallas as pl
from jax.experimental.pallas import tpu as pltpu

N_DEV = 4
NT = 256
CHUNK_TILES = 2

_SKIP_COMM = os.environ.get("SKIP_COMM") == "1"


def kernel(x, w_mat):
    x = x.astype(jnp.bfloat16)
    m_per, k = x.shape
    _, n = w_mat.shape
    n_per = n // N_DEV
    tpb = n_per // NT
    n_tiles = N_DEV * tpb
    nc = NT * CHUNK_TILES
    n_chunks = n_per // nc

    def body(x_ref, w_ref, out_ref, send_buf, w_buf, w_sems, send_sems, recv_sems):
        my = lax.axis_index("i")

        barrier = pltpu.get_barrier_semaphore()
        for off in (1, 2, 3):
            peer = lax.rem(my + off, N_DEV)
            pl.semaphore_signal(
                barrier, inc=1,
                device_id=(peer,), device_id_type=pl.DeviceIdType.MESH,
            )
        pl.semaphore_wait(barrier, N_DEV - 1)

        def w_copy(idx, slot):
            bi = idx // tpb
            t = lax.rem(idx, tpb)
            dest = lax.rem(my + bi + 1, N_DEV)
            col = dest * n_per + t * NT
            return pltpu.make_async_copy(
                w_ref.at[:, pl.ds(col, NT)], w_buf.at[slot], w_sems.at[slot]
            )

        w_copy(jnp.int32(0), 0).start()
        w_copy(jnp.int32(1), 1).start()

        def tile(idx, slot):
            bi = idx // tpb
            t = lax.rem(idx, tpb)
            dest = lax.rem(my + bi + 1, N_DEV)
            own = bi == N_DEV - 1
            sb = jnp.minimum(bi, N_DEV - 2)

            w_copy(idx, slot).wait()
            acc = jnp.dot(
                x_ref[...], w_buf[slot].astype(jnp.bfloat16),
                preferred_element_type=jnp.float32,
            ).astype(jnp.bfloat16)

            @pl.when(own)
            def _():
                out_ref[pl.ds(my * m_per, m_per), pl.ds(t * NT, NT)] = acc

            @pl.when(jnp.logical_not(own))
            def _():
                send_buf[sb, :, pl.ds(t * NT, NT)] = acc

            @pl.when(jnp.logical_and(
                jnp.logical_not(own),
                (lax.rem(t, CHUNK_TILES) == CHUNK_TILES - 1) & (not _SKIP_COMM)))
            def _():
                qtr = t // CHUNK_TILES
                pltpu.make_async_remote_copy(
                    src_ref=send_buf.at[sb, :, pl.ds(qtr * nc, nc)],
                    dst_ref=out_ref.at[pl.ds(my * m_per, m_per), pl.ds(qtr * nc, nc)],
                    send_sem=send_sems.at[sb, qtr],
                    recv_sem=recv_sems.at[my, qtr],
                    device_id=(dest,),
                    device_id_type=pl.DeviceIdType.MESH,
                ).start()

            @pl.when(idx + 2 < n_tiles)
            def _():
                w_copy(idx + 2, slot).start()

        def pair_step(j, carry):
            tile(j * 2, 0)
            tile(j * 2 + 1, 1)
            return carry

        lax.fori_loop(0, n_tiles // 2, pair_step, 0)

        if not _SKIP_COMM:
            for sb in range(N_DEV - 1):
                for qtr in range(n_chunks):
                    pltpu.make_async_remote_copy(
                        src_ref=send_buf.at[sb, :, pl.ds(qtr * nc, nc)],
                        dst_ref=out_ref.at[pl.ds(my * m_per, m_per), pl.ds(qtr * nc, nc)],
                        send_sem=send_sems.at[sb, qtr],
                        recv_sem=recv_sems.at[my, qtr],
                        device_id=(lax.rem(my + 1, N_DEV),),
                        device_id_type=pl.DeviceIdType.MESH,
                    ).wait_send()

            for off in (3, 2, 1):
                src = lax.rem(my + off, N_DEV)
                for qtr in range(n_chunks):
                    pltpu.make_async_remote_copy(
                        src_ref=send_buf.at[0, :, pl.ds(qtr * nc, nc)],
                        dst_ref=out_ref.at[pl.ds(src * m_per, m_per), pl.ds(qtr * nc, nc)],
                        send_sem=send_sems.at[0, qtr],
                        recv_sem=recv_sems.at[src, qtr],
                        device_id=(src,),
                        device_id_type=pl.DeviceIdType.MESH,
                    ).wait_recv()

    return pl.pallas_call(
        body,
        out_shape=jax.ShapeDtypeStruct((N_DEV * m_per, n_per), jnp.bfloat16),
        in_specs=[
            pl.BlockSpec(memory_space=pltpu.MemorySpace.VMEM),
            pl.BlockSpec(memory_space=pl.ANY),
        ],
        out_specs=pl.BlockSpec(memory_space=pltpu.MemorySpace.VMEM),
        scratch_shapes=[
            pltpu.VMEM((N_DEV - 1, m_per, n_per), jnp.bfloat16),
            pltpu.VMEM((2, k, NT), jnp.float32),
            pltpu.SemaphoreType.DMA((2,)),
            pltpu.SemaphoreType.DMA((N_DEV - 1, 4)),
            pltpu.SemaphoreType.DMA((N_DEV, 4)),
        ],
        compiler_params=pltpu.CompilerParams(
            collective_id=0,
            vmem_limit_bytes=38 * 1024 * 1024,
        ),
    )(x, w_mat)


# device time: 149600 ns/iter; 1.2094x vs baseline; 1.2094x over previous
import os

import jax
import jax.numpy as jnp
from jax import lax
from jax.experimental import pallas as pl
from jax.experimental.pallas import tpu as pltpu

N_DEV = 4
NT = 256
CHUNK_TILES = 2

_SKIP_COMM = os.environ.get("SKIP_COMM") == "1"


def kernel(x, w_mat):
    x = x.astype(jnp.bfloat16)
    m_per, k = x.shape
    _, n = w_mat.shape
    n_per = n // N_DEV
    tpb = n_per // NT
    n_tiles = N_DEV * tpb
    nc = NT * CHUNK_TILES
    n_chunks = n_per // nc

    def body(x_ref, w_ref, out_ref, send_buf, w_buf, w_sems, send_sems, recv_sems):
        my = lax.axis_index("i")

        barrier = pltpu.get_barrier_semaphore()
        for off in (1, 2, 3):
            peer = lax.rem(my + off, N_DEV)
            pl.semaphore_signal(
                barrier, inc=1,
                device_id=(peer,), device_id_type=pl.DeviceIdType.MESH,
            )
        pl.semaphore_wait(barrier, N_DEV - 1)

        def w_copy(idx, slot):
            bi = idx // tpb
            t = lax.rem(idx, tpb)
            dest = lax.rem(my + bi + 1, N_DEV)
            col = dest * n_per + t * NT
            return pltpu.make_async_copy(
                w_ref.at[:, pl.ds(col, NT)], w_buf.at[slot], w_sems.at[slot]
            )

        w_copy(jnp.int32(0), 0).start()
        w_copy(jnp.int32(1), 1).start()

        def tile_step(idx, carry):
            bi = idx // tpb
            t = lax.rem(idx, tpb)
            dest = lax.rem(my + bi + 1, N_DEV)
            slot = lax.rem(idx, 2)
            own = bi == N_DEV - 1
            sb = jnp.minimum(bi, N_DEV - 2)

            w_copy(idx, slot).wait()
            acc = jnp.dot(
                x_ref[...], w_buf[slot].astype(jnp.bfloat16),
                preferred_element_type=jnp.float32,
            ).astype(jnp.bfloat16)

            @pl.when(own)
            def _():
                out_ref[pl.ds(my * m_per, m_per), pl.ds(t * NT, NT)] = acc

            @pl.when(jnp.logical_not(own))
            def _():
                send_buf[sb, :, pl.ds(t * NT, NT)] = acc

            @pl.when(jnp.logical_and(
                jnp.logical_not(own),
                (lax.rem(t, CHUNK_TILES) == CHUNK_TILES - 1) & (not _SKIP_COMM)))
            def _():
                qtr = t // CHUNK_TILES
                pltpu.make_async_remote_copy(
                    src_ref=send_buf.at[sb, :, pl.ds(qtr * nc, nc)],
                    dst_ref=out_ref.at[pl.ds(my * m_per, m_per), pl.ds(qtr * nc, nc)],
                    send_sem=send_sems.at[sb, qtr],
                    recv_sem=recv_sems.at[my, qtr],
                    device_id=(dest,),
                    device_id_type=pl.DeviceIdType.MESH,
                ).start()

            @pl.when(idx + 2 < n_tiles)
            def _():
                w_copy(idx + 2, slot).start()

            return carry

        lax.fori_loop(0, n_tiles, tile_step, 0)

        if not _SKIP_COMM:
            for sb in range(N_DEV - 1):
                for qtr in range(n_chunks):
                    pltpu.make_async_remote_copy(
                        src_ref=send_buf.at[sb, :, pl.ds(qtr * nc, nc)],
                        dst_ref=out_ref.at[pl.ds(my * m_per, m_per), pl.ds(qtr * nc, nc)],
                        send_sem=send_sems.at[sb, qtr],
                        recv_sem=recv_sems.at[my, qtr],
                        device_id=(lax.rem(my + 1, N_DEV),),
                        device_id_type=pl.DeviceIdType.MESH,
                    ).wait_send()

            for off in (3, 2, 1):
                src = lax.rem(my + off, N_DEV)
                for qtr in range(n_chunks):
                    pltpu.make_async_remote_copy(
                        src_ref=send_buf.at[0, :, pl.ds(qtr * nc, nc)],
                        dst_ref=out_ref.at[pl.ds(src * m_per, m_per), pl.ds(qtr * nc, nc)],
                        send_sem=send_sems.at[0, qtr],
                        recv_sem=recv_sems.at[src, qtr],
                        device_id=(src,),
                        device_id_type=pl.DeviceIdType.MESH,
                    ).wait_recv()

    return pl.pallas_call(
        body,
        out_shape=jax.ShapeDtypeStruct((N_DEV * m_per, n_per), jnp.bfloat16),
        in_specs=[
            pl.BlockSpec(memory_space=pltpu.MemorySpace.VMEM),
            pl.BlockSpec(memory_space=pl.ANY),
        ],
        out_specs=pl.BlockSpec(memory_space=pltpu.MemorySpace.VMEM),
        scratch_shapes=[
            pltpu.VMEM((N_DEV - 1, m_per, n_per), jnp.bfloat16),
            pltpu.VMEM((2, k, NT), jnp.float32),
            pltpu.SemaphoreType.DMA((2,)),
            pltpu.SemaphoreType.DMA((N_DEV - 1, 4)),
            pltpu.SemaphoreType.DMA((N_DEV, 4)),
        ],
        compiler_params=pltpu.CompilerParams(
            collective_id=0,
            vmem_limit_bytes=38 * 1024 * 1024,
        ),
    )(x, w_mat)


# device time: 133899 ns/iter; 1.3512x vs baseline; 1.1173x over previous
import os

import jax
import jax.numpy as jnp
from jax import lax
from jax.experimental import pallas as pl
from jax.experimental.pallas import tpu as pltpu

N_DEV = 4
NT = 256
CHUNK_TILES = 2

_SKIP_COMM = os.environ.get("SKIP_COMM") == "1"


def kernel(x, w_mat):
    x = x.astype(jnp.bfloat16)
    m_per, k = x.shape
    _, n = w_mat.shape
    n_per = n // N_DEV
    tpb = n_per // NT
    n_tiles = N_DEV * tpb
    nc = NT * CHUNK_TILES
    n_chunks = n_per // nc

    def body(x_ref, w_ref, out_ref, send_buf, w_buf, w_sems, send_sems, recv_sems):
        my = lax.axis_index("i")

        barrier = pltpu.get_barrier_semaphore()
        for off in (1, 2, 3):
            peer = lax.rem(my + off, N_DEV)
            pl.semaphore_signal(
                barrier, inc=1,
                device_id=(peer,), device_id_type=pl.DeviceIdType.MESH,
            )
        pl.semaphore_wait(barrier, N_DEV - 1)

        send_tiles = (N_DEV - 1) * tpb

        def tile_params(idx):
            c = idx // CHUNK_TILES
            u = lax.rem(idx, CHUNK_TILES)
            send_phase = idx < send_tiles
            bi = jnp.where(send_phase, lax.rem(c, N_DEV - 1), N_DEV - 1)
            qtr = (c // (N_DEV - 1)) * CHUNK_TILES + u
            t = jnp.where(send_phase, qtr, idx - send_tiles)
            dest = lax.rem(my + bi + 1, N_DEV)
            return bi, t, u, dest, send_phase

        def w_copy(idx, slot):
            bi, t, u, dest, send_phase = tile_params(idx)
            col = dest * n_per + t * NT
            return pltpu.make_async_copy(
                w_ref.at[:, pl.ds(col, NT)], w_buf.at[slot], w_sems.at[slot]
            )

        w_copy(jnp.int32(0), 0).start()
        w_copy(jnp.int32(1), 1).start()

        def tile_step(idx, carry):
            bi, t, u, dest, send_phase = tile_params(idx)
            slot = lax.rem(idx, 2)
            own = jnp.logical_not(send_phase)
            sb = jnp.minimum(bi, N_DEV - 2)

            w_copy(idx, slot).wait()
            acc = jnp.dot(
                x_ref[...], w_buf[slot].astype(jnp.bfloat16),
                preferred_element_type=jnp.float32,
            ).astype(jnp.bfloat16)

            @pl.when(own)
            def _():
                out_ref[pl.ds(my * m_per, m_per), pl.ds(t * NT, NT)] = acc

            @pl.when(send_phase)
            def _():
                send_buf[sb, :, pl.ds(t * NT, NT)] = acc

            @pl.when(jnp.logical_and(
                send_phase,
                (u == CHUNK_TILES - 1) & (not _SKIP_COMM)))
            def _():
                qtr = t // CHUNK_TILES
                pltpu.make_async_remote_copy(
                    src_ref=send_buf.at[sb, :, pl.ds(qtr * nc, nc)],
                    dst_ref=out_ref.at[pl.ds(my * m_per, m_per), pl.ds(qtr * nc, nc)],
                    send_sem=send_sems.at[sb, qtr],
                    recv_sem=recv_sems.at[my, qtr],
                    device_id=(dest,),
                    device_id_type=pl.DeviceIdType.MESH,
                ).start()

            @pl.when(idx + 2 < n_tiles)
            def _():
                w_copy(idx + 2, slot).start()

            return carry

        lax.fori_loop(0, n_tiles, tile_step, 0)

        if not _SKIP_COMM:
            for sb in range(N_DEV - 1):
                for qtr in range(n_chunks):
                    pltpu.make_async_remote_copy(
                        src_ref=send_buf.at[sb, :, pl.ds(qtr * nc, nc)],
                        dst_ref=out_ref.at[pl.ds(my * m_per, m_per), pl.ds(qtr * nc, nc)],
                        send_sem=send_sems.at[sb, qtr],
                        recv_sem=recv_sems.at[my, qtr],
                        device_id=(lax.rem(my + 1, N_DEV),),
                        device_id_type=pl.DeviceIdType.MESH,
                    ).wait_send()

            for qtr in range(n_chunks):
                for off in (3, 2, 1):
                    src = lax.rem(my + off, N_DEV)
                    pltpu.make_async_remote_copy(
                        src_ref=send_buf.at[0, :, pl.ds(qtr * nc, nc)],
                        dst_ref=out_ref.at[pl.ds(src * m_per, m_per), pl.ds(qtr * nc, nc)],
                        send_sem=send_sems.at[0, qtr],
                        recv_sem=recv_sems.at[src, qtr],
                        device_id=(src,),
                        device_id_type=pl.DeviceIdType.MESH,
                    ).wait_recv()

    return pl.pallas_call(
        body,
        out_shape=jax.ShapeDtypeStruct((N_DEV * m_per, n_per), jnp.bfloat16),
        in_specs=[
            pl.BlockSpec(memory_space=pltpu.MemorySpace.VMEM),
            pl.BlockSpec(memory_space=pl.ANY),
        ],
        out_specs=pl.BlockSpec(memory_space=pltpu.MemorySpace.VMEM),
        scratch_shapes=[
            pltpu.VMEM((N_DEV - 1, m_per, n_per), jnp.bfloat16),
            pltpu.VMEM((2, k, NT), jnp.float32),
            pltpu.SemaphoreType.DMA((2,)),
            pltpu.SemaphoreType.DMA((N_DEV - 1, 4)),
            pltpu.SemaphoreType.DMA((N_DEV, 4)),
        ],
        compiler_params=pltpu.CompilerParams(
            collective_id=0,
            vmem_limit_bytes=38 * 1024 * 1024,
        ),
    )(x, w_mat)


# device time: 128861 ns/iter; 1.4041x vs baseline; 1.0391x over previous
import os

import jax
import jax.numpy as jnp
from jax import lax
from jax.experimental import pallas as pl
from jax.experimental.pallas import tpu as pltpu

N_DEV = 4
NT = 256
CHUNK_TILES = 1

_SKIP_COMM = os.environ.get("SKIP_COMM") == "1"


def kernel(x, w_mat):
    x = x.astype(jnp.bfloat16)
    m_per, k = x.shape
    _, n = w_mat.shape
    n_per = n // N_DEV
    tpb = n_per // NT
    n_tiles = N_DEV * tpb
    nc = NT * CHUNK_TILES
    n_chunks = n_per // nc

    def body(x_ref, w_ref, out_ref, send_buf, w_buf, w_sems, send_sems, recv_sems):
        my = lax.axis_index("i")

        barrier = pltpu.get_barrier_semaphore()
        for off in (1, 2, 3):
            peer = lax.rem(my + off, N_DEV)
            pl.semaphore_signal(
                barrier, inc=1,
                device_id=(peer,), device_id_type=pl.DeviceIdType.MESH,
            )
        pl.semaphore_wait(barrier, N_DEV - 1)

        send_tiles = (N_DEV - 1) * tpb

        def tile_params(idx):
            c = idx // CHUNK_TILES
            u = lax.rem(idx, CHUNK_TILES)
            send_phase = idx < send_tiles
            bi = jnp.where(send_phase, lax.rem(c, N_DEV - 1), N_DEV - 1)
            qtr = (c // (N_DEV - 1)) * CHUNK_TILES + u
            t = jnp.where(send_phase, qtr, idx - send_tiles)
            dest = lax.rem(my + bi + 1, N_DEV)
            return bi, t, u, dest, send_phase

        def w_copy(idx, slot):
            bi, t, u, dest, send_phase = tile_params(idx)
            col = dest * n_per + t * NT
            return pltpu.make_async_copy(
                w_ref.at[:, pl.ds(col, NT)], w_buf.at[slot], w_sems.at[slot]
            )

        w_copy(jnp.int32(0), 0).start()
        w_copy(jnp.int32(1), 1).start()

        def tile_step(idx, carry):
            bi, t, u, dest, send_phase = tile_params(idx)
            slot = lax.rem(idx, 2)
            own = jnp.logical_not(send_phase)
            sb = jnp.minimum(bi, N_DEV - 2)

            w_copy(idx, slot).wait()
            acc = jnp.dot(
                x_ref[...], w_buf[slot].astype(jnp.bfloat16),
                preferred_element_type=jnp.float32,
            ).astype(jnp.bfloat16)

            @pl.when(own)
            def _():
                out_ref[pl.ds(my * m_per, m_per), pl.ds(t * NT, NT)] = acc

            @pl.when(send_phase)
            def _():
                send_buf[sb, :, pl.ds(t * NT, NT)] = acc

            @pl.when(jnp.logical_and(
                send_phase,
                (u == CHUNK_TILES - 1) & (not _SKIP_COMM)))
            def _():
                qtr = t // CHUNK_TILES
                pltpu.make_async_remote_copy(
                    src_ref=send_buf.at[sb, :, pl.ds(qtr * nc, nc)],
                    dst_ref=out_ref.at[pl.ds(my * m_per, m_per), pl.ds(qtr * nc, nc)],
                    send_sem=send_sems.at[sb, qtr],
                    recv_sem=recv_sems.at[my, qtr],
                    device_id=(dest,),
                    device_id_type=pl.DeviceIdType.MESH,
                ).start()

            @pl.when(idx + 2 < n_tiles)
            def _():
                w_copy(idx + 2, slot).start()

            return carry

        lax.fori_loop(0, n_tiles, tile_step, 0)

        if not _SKIP_COMM:
            for sb in range(N_DEV - 1):
                for qtr in range(n_chunks):
                    pltpu.make_async_remote_copy(
                        src_ref=send_buf.at[sb, :, pl.ds(qtr * nc, nc)],
                        dst_ref=out_ref.at[pl.ds(my * m_per, m_per), pl.ds(qtr * nc, nc)],
                        send_sem=send_sems.at[sb, qtr],
                        recv_sem=recv_sems.at[my, qtr],
                        device_id=(lax.rem(my + 1, N_DEV),),
                        device_id_type=pl.DeviceIdType.MESH,
                    ).wait_send()

            for qtr in range(n_chunks):
                for off in (3, 2, 1):
                    src = lax.rem(my + off, N_DEV)
                    pltpu.make_async_remote_copy(
                        src_ref=send_buf.at[0, :, pl.ds(qtr * nc, nc)],
                        dst_ref=out_ref.at[pl.ds(src * m_per, m_per), pl.ds(qtr * nc, nc)],
                        send_sem=send_sems.at[0, qtr],
                        recv_sem=recv_sems.at[src, qtr],
                        device_id=(src,),
                        device_id_type=pl.DeviceIdType.MESH,
                    ).wait_recv()

    return pl.pallas_call(
        body,
        out_shape=jax.ShapeDtypeStruct((N_DEV * m_per, n_per), jnp.bfloat16),
        in_specs=[
            pl.BlockSpec(memory_space=pltpu.MemorySpace.VMEM),
            pl.BlockSpec(memory_space=pl.ANY),
        ],
        out_specs=pl.BlockSpec(memory_space=pltpu.MemorySpace.VMEM),
        scratch_shapes=[
            pltpu.VMEM((N_DEV - 1, m_per, n_per), jnp.bfloat16),
            pltpu.VMEM((2, k, NT), jnp.float32),
            pltpu.SemaphoreType.DMA((2,)),
            pltpu.SemaphoreType.DMA((N_DEV - 1, n_chunks)),
            pltpu.SemaphoreType.DMA((N_DEV, n_chunks)),
        ],
        compiler_params=pltpu.CompilerParams(
            collective_id=0,
            vmem_limit_bytes=38 * 1024 * 1024,
        ),
    )(x, w_mat)
